# baseline (speedup 1.0000x reference)
import functools

import jax
import jax.numpy as jnp
from jax import lax
from jax.experimental import pallas as pl
from jax.experimental.pallas import tpu as pltpu

_N1 = 25000
_N2 = 6250


def _mm2_relu_kernel(a_ref, b_ref, wa_ref, wb_ref, bias_ref, o_ref):
    acc = jnp.dot(a_ref[...], wa_ref[...], preferred_element_type=jnp.float32)
    acc = acc + jnp.dot(b_ref[...], wb_ref[...], preferred_element_type=jnp.float32)
    o_ref[...] = jnp.maximum(acc + bias_ref[...], 0.0)


def _mm2_relu(a, b, wa, wb, bias, bn=2048, interpret=False):
    n = a.shape[0]
    grid = (pl.cdiv(n, bn),)
    return pl.pallas_call(
        _mm2_relu_kernel,
        grid=grid,
        in_specs=[
            pl.BlockSpec((bn, a.shape[1]), lambda i: (i, 0)),
            pl.BlockSpec((bn, b.shape[1]), lambda i: (i, 0)),
            pl.BlockSpec(wa.shape, lambda i: (0, 0)),
            pl.BlockSpec(wb.shape, lambda i: (0, 0)),
            pl.BlockSpec((1, bias.shape[0]), lambda i: (0, 0)),
        ],
        out_specs=pl.BlockSpec((bn, wa.shape[1]), lambda i: (i, 0)),
        out_shape=jax.ShapeDtypeStruct((n, wa.shape[1]), jnp.float32),
        interpret=interpret,
    )(a, b, wa, wb, bias.reshape(1, -1))


def kernel(x, pos, cluster0, cluster1, Wp0, bp0, Wp1, bp1, Wm1, bm1, Wm0, bm0):
    D = x.shape[1]
    h0 = _mm2_relu(x, pos, Wp0[:D], Wp0[D:], bp0)
    x1 = jax.ops.segment_max(h0, cluster0, num_segments=_N1)
    x1 = jnp.where(jnp.isfinite(x1), x1, 0.0)
    cnt = jax.ops.segment_sum(jnp.ones((cluster0.shape[0], 1), jnp.float32),
                              cluster0, num_segments=_N1)
    pos1 = jax.ops.segment_sum(pos, cluster0, num_segments=_N1) / jnp.maximum(cnt, 1.0)

    h1 = _mm2_relu(x1, pos1, Wp1[:D], Wp1[D:], bp1)
    x2 = jax.ops.segment_max(h1, cluster1, num_segments=_N2)
    x2 = jnp.where(jnp.isfinite(x2), x2, 0.0)

    h2 = _mm2_relu(jnp.take(x2, cluster1, axis=0), x1, Wm1[:D], Wm1[D:], bm1)
    out = _mm2_relu(jnp.take(h2, cluster0, axis=0), x, Wm0[:D], Wm0[D:], bm0)
    return out
